# baseline (speedup 1.0000x reference)
import functools

import jax
import jax.numpy as jnp
from jax import lax
from jax.experimental import pallas as pl
from jax.experimental.pallas import tpu as pltpu
from jax.experimental.pallas import tpu_sc as plsc

GAMMA = 12.0
BATCH = 4096
NEG = 200
DIM = 64

_info = plsc.get_sparse_core_info()
NC, NS, L = _info.num_cores, _info.num_subcores, _info.num_lanes
NW = NC * NS
BPW = BATCH // NW

_GROUP_OFFS = list(range(0, NEG - L, L)) + [NEG - L]


def _score_kernel(pos_hbm, neg_hbm, ent_hbm, rel_hbm, out_hbm,
                  pos_v, hidx_v, ridx_v, h_v, r_v, nidx_v,
                  t0_v, t1_v, out_v, sem0, sem1, semh, semr):
    wid = lax.axis_index("s") * NC + lax.axis_index("c")
    base = wid * BPW
    iota = lax.iota(jnp.int32, L)

    pltpu.sync_copy(pos_hbm.at[pl.ds(base, BPW), :], pos_v)
    pltpu.sync_copy(neg_hbm.at[pl.ds(base, BPW), :], nidx_v)

    for i in range(BPW // L):
        rows = iota + (i * L)
        hvals = plsc.load_gather(pos_v, [rows, jnp.zeros((L,), jnp.int32)])
        rvals = plsc.load_gather(pos_v, [rows, jnp.ones((L,), jnp.int32)])
        hidx_v[pl.ds(i * L, L)] = hvals
        ridx_v[pl.ds(i * L, L)] = rvals

    ch = pltpu.async_copy(ent_hbm.at[hidx_v], h_v, semh)
    cr = pltpu.async_copy(rel_hbm.at[ridx_v], r_v, semr)
    ch.wait()
    cr.wait()

    def hr_body(b, carry):
        for s in range(DIM // L):
            sl = pl.ds(s * L, L)
            h_v[b, sl] = h_v[b, sl] + r_v[b, sl]
        return carry
    lax.fori_loop(0, BPW, hr_body, 0)

    def start_tail(b, t_v):
        pltpu.async_copy(ent_hbm.at[nidx_v.at[b, pl.ds(0, 128)]],
                         t_v.at[pl.ds(0, 128), :], sem0)
        pltpu.async_copy(ent_hbm.at[nidx_v.at[b, pl.ds(128, NEG - 128)]],
                         t_v.at[pl.ds(128, NEG - 128), :], sem1)

    def wait_tail(b, t_v):
        pltpu.make_async_copy(ent_hbm.at[nidx_v.at[b, pl.ds(0, 128)]],
                              t_v.at[pl.ds(0, 128), :], sem0).wait()
        pltpu.make_async_copy(ent_hbm.at[nidx_v.at[b, pl.ds(128, NEG - 128)]],
                              t_v.at[pl.ds(128, NEG - 128), :], sem1).wait()

    def compute_b(b, t_v):
        def s_body(s, accs):
            hv = h_v[b, pl.ds(s * L, L)]
            d0 = s * L
            for dd in range(L):
                hs = hv[dd]
                col = jnp.full((L,), d0 + dd, jnp.int32)
                new = []
                for g, off in enumerate(_GROUP_OFFS):
                    rows = iota + off
                    tv = plsc.load_gather(t_v, [rows, col])
                    new.append(accs[g] + jnp.abs(hs - tv))
                accs = tuple(new)
            return accs

        zeros = jnp.zeros((L,), jnp.float32)
        accs = lax.fori_loop(0, DIM // L, s_body,
                             tuple(zeros for _ in _GROUP_OFFS))
        for g, off in enumerate(_GROUP_OFFS):
            out_v[b, pl.ds(off, L)] = GAMMA - accs[g]

    start_tail(0, t0_v)

    def b_body(i, carry):
        b = i * 2
        start_tail(b + 1, t1_v)
        wait_tail(b, t0_v)
        compute_b(b, t0_v)

        @pl.when(b + 2 < BPW)
        def _():
            start_tail(b + 2, t0_v)
        wait_tail(b + 1, t1_v)
        compute_b(b + 1, t1_v)
        return carry

    lax.fori_loop(0, BPW // 2, b_body, 0)

    pltpu.sync_copy(out_v, out_hbm.at[pl.ds(base, BPW), :])


@jax.jit
def _kge_score(positive_sample, negative_sample, entity_embedding,
               relation_embedding):
    mesh = plsc.VectorSubcoreMesh(core_axis_name="c", subcore_axis_name="s")
    run = functools.partial(
        pl.kernel,
        out_type=jax.ShapeDtypeStruct((BATCH, NEG), jnp.float32),
        mesh=mesh,
        compiler_params=pltpu.CompilerParams(
            needs_layout_passes=False, use_tc_tiling_on_sc=False),
        scratch_types=[
            pltpu.VMEM((BPW, 3), jnp.int32),
            pltpu.VMEM((BPW,), jnp.int32),
            pltpu.VMEM((BPW,), jnp.int32),
            pltpu.VMEM((BPW, DIM), jnp.float32),
            pltpu.VMEM((BPW, DIM), jnp.float32),
            pltpu.VMEM((BPW, NEG), jnp.int32),
            pltpu.VMEM((NEG, DIM), jnp.float32),
            pltpu.VMEM((NEG, DIM), jnp.float32),
            pltpu.VMEM((BPW, NEG), jnp.float32),
            pltpu.SemaphoreType.DMA,
            pltpu.SemaphoreType.DMA,
            pltpu.SemaphoreType.DMA,
            pltpu.SemaphoreType.DMA,
        ],
    )(_score_kernel)
    return run(positive_sample, negative_sample, entity_embedding,
               relation_embedding)


def kernel(positive_sample, negative_sample, entity_embedding,
           relation_embedding):
    return _kge_score(positive_sample, negative_sample, entity_embedding,
                      relation_embedding)


# speedup vs baseline: 2.1299x; 2.1299x over previous
import functools

import numpy as np

import jax
import jax.numpy as jnp
from jax import lax
from jax.experimental import pallas as pl
from jax.experimental.pallas import tpu as pltpu
from jax.experimental.pallas import tpu_sc as plsc

GAMMA = 12.0
BATCH = 4096
NEG = 200
DIM = 64

_info = plsc.get_sparse_core_info()
NC, NS, L = _info.num_cores, _info.num_subcores, _info.num_lanes
NW = NC * NS
BPW = BATCH // NW

_GROUP_OFFS = list(range(0, NEG - L, L)) + [NEG - L]

_BITREV = [0, 8, 4, 12, 2, 10, 6, 14, 1, 9, 5, 13, 3, 11, 7, 15]


def _score_kernel(pos_hbm, neg_hbm, ent_hbm, rel_hbm, out_hbm,
                  pos_v, hidx_v, ridx_v, h_v, r_v, nidx_v,
                  t0_v, t1_v, out_v, sem0, sem1, semh, semr):
    wid = lax.axis_index("s") * NC + lax.axis_index("c")
    base = wid * BPW
    iota = lax.iota(jnp.int32, L)

    pltpu.sync_copy(pos_hbm.at[pl.ds(base, BPW), :], pos_v)
    pltpu.sync_copy(neg_hbm.at[pl.ds(base, BPW), :], nidx_v)

    for i in range(BPW // L):
        rows = iota + (i * L)
        hvals = plsc.load_gather(pos_v, [rows, jnp.zeros((L,), jnp.int32)])
        rvals = plsc.load_gather(pos_v, [rows, jnp.ones((L,), jnp.int32)])
        hidx_v[pl.ds(i * L, L)] = hvals
        ridx_v[pl.ds(i * L, L)] = rvals

    ch = pltpu.async_copy(ent_hbm.at[hidx_v], h_v, semh)
    cr = pltpu.async_copy(rel_hbm.at[ridx_v], r_v, semr)
    ch.wait()
    cr.wait()

    def hr_body(b, carry):
        for s in range(DIM // L):
            sl = pl.ds(s * L, L)
            h_v[b, sl] = h_v[b, sl] + r_v[b, sl]
        return carry
    lax.fori_loop(0, BPW, hr_body, 0)

    def start_tail(b, t_v):
        pltpu.async_copy(ent_hbm.at[nidx_v.at[b, pl.ds(0, 128)]],
                         t_v.at[pl.ds(0, 128), :], sem0)
        pltpu.async_copy(ent_hbm.at[nidx_v.at[b, pl.ds(128, NEG - 128)]],
                         t_v.at[pl.ds(128, NEG - 128), :], sem1)

    def wait_tail(b, t_v):
        pltpu.make_async_copy(ent_hbm.at[nidx_v.at[b, pl.ds(0, 128)]],
                              t_v.at[pl.ds(0, 128), :], sem0).wait()
        pltpu.make_async_copy(ent_hbm.at[nidx_v.at[b, pl.ds(128, NEG - 128)]],
                              t_v.at[pl.ds(128, NEG - 128), :], sem1).wait()

    masks = {s: (iota & s) == 0 for s in (1, 2, 4, 8)}
    rollm = {s: (iota - s) % L for s in (1, 2, 4, 8)}
    rollp = {s: (iota + s) % L for s in (1, 2, 4, 8)}
    def combine(a, bb, s):
        rb = bb.at[rollm[s]].get(mode="promise_in_bounds")
        ra = a.at[rollp[s]].get(mode="promise_in_bounds")
        return jnp.where(masks[s], a, rb) + jnp.where(masks[s], ra, bb)

    def compute_b(b, t_v):
        hr = [h_v[b, pl.ds(s * L, L)] for s in range(DIM // L)]

        def g_body(g, carry):
            off = lax.min(g * L, NEG - L)
            parts = []
            for k in _BITREV:
                row = off + k
                p01 = (jnp.abs(hr[0] - t_v[row, pl.ds(0, L)])
                       + jnp.abs(hr[1] - t_v[row, pl.ds(L, L)]))
                p23 = (jnp.abs(hr[2] - t_v[row, pl.ds(2 * L, L)])
                       + jnp.abs(hr[3] - t_v[row, pl.ds(3 * L, L)]))
                parts.append(p01 + p23)
            w = [combine(parts[2 * i], parts[2 * i + 1], 8) for i in range(8)]
            x = [combine(w[2 * i], w[2 * i + 1], 4) for i in range(4)]
            y = [combine(x[2 * i], x[2 * i + 1], 2) for i in range(2)]
            z = combine(y[0], y[1], 1)
            out_v[b, pl.ds(off, L)] = GAMMA - z
            return carry

        lax.fori_loop(0, len(_GROUP_OFFS), g_body, 0)

    start_tail(0, t0_v)

    def b_body(i, carry):
        b = i * 2
        start_tail(b + 1, t1_v)
        wait_tail(b, t0_v)
        compute_b(b, t0_v)

        @pl.when(b + 2 < BPW)
        def _():
            start_tail(b + 2, t0_v)
        wait_tail(b + 1, t1_v)
        compute_b(b + 1, t1_v)
        return carry

    lax.fori_loop(0, BPW // 2, b_body, 0)

    pltpu.sync_copy(out_v, out_hbm.at[pl.ds(base, BPW), :])


@jax.jit
def _kge_score(positive_sample, negative_sample, entity_embedding,
               relation_embedding):
    mesh = plsc.VectorSubcoreMesh(core_axis_name="c", subcore_axis_name="s")
    run = functools.partial(
        pl.kernel,
        out_type=jax.ShapeDtypeStruct((BATCH, NEG), jnp.float32),
        mesh=mesh,
        compiler_params=pltpu.CompilerParams(
            needs_layout_passes=False, use_tc_tiling_on_sc=False),
        scratch_types=[
            pltpu.VMEM((BPW, 3), jnp.int32),
            pltpu.VMEM((BPW,), jnp.int32),
            pltpu.VMEM((BPW,), jnp.int32),
            pltpu.VMEM((BPW, DIM), jnp.float32),
            pltpu.VMEM((BPW, DIM), jnp.float32),
            pltpu.VMEM((BPW, NEG), jnp.int32),
            pltpu.VMEM((NEG, DIM), jnp.float32),
            pltpu.VMEM((NEG, DIM), jnp.float32),
            pltpu.VMEM((BPW, NEG), jnp.float32),
            pltpu.SemaphoreType.DMA,
            pltpu.SemaphoreType.DMA,
            pltpu.SemaphoreType.DMA,
            pltpu.SemaphoreType.DMA,
        ],
    )(_score_kernel)
    return run(positive_sample, negative_sample, entity_embedding,
               relation_embedding)


def kernel(positive_sample, negative_sample, entity_embedding,
           relation_embedding):
    return _kge_score(positive_sample, negative_sample, entity_embedding,
                      relation_embedding)
